# baseline (speedup 1.0000x reference)
import functools

import jax
import jax.numpy as jnp
from jax import lax
from jax.experimental import pallas as pl
from jax.experimental.pallas import tpu as pltpu
from jax.experimental.pallas import tpu_sc as plsc

N_TOK = 8192
D = 4096
N_IDX = 4096
NC = 2
NS = 16
NW = NC * NS
PER_W = N_IDX // NW
G = 8
N_CHUNK = PER_W // G
LANES = 16
D_VECS = D // LANES
COLS_W = D // NW

_mesh = plsc.VectorSubcoreMesh(core_axis_name="c", subcore_axis_name="s")
_params = pltpu.CompilerParams(needs_layout_passes=False)


@functools.partial(
    pl.kernel,
    mesh=_mesh,
    compiler_params=_params,
    out_type=[
        jax.ShapeDtypeStruct((NW, D), jnp.float32),
        jax.ShapeDtypeStruct((NW, LANES), jnp.float32),
    ],
    scratch_types=[
        pltpu.VMEM((PER_W,), jnp.int32),
        pltpu.VMEM((N_TOK,), jnp.float32),
        pltpu.VMEM((PER_W,), jnp.float32),
        pltpu.VMEM((G, D), jnp.float32),
        pltpu.VMEM((D,), jnp.float32),
        pltpu.VMEM((LANES,), jnp.float32),
        pltpu.SemaphoreType.DMA,
    ],
)
def _partial_sums(tokens_hbm, s_hbm, idx_hbm, acc_out, den_out,
                  idx_v, s_v, ssel_v, rows_v, acc_v, den_v, sem):
    wid = lax.axis_index("s") * NC + lax.axis_index("c")
    base = wid * PER_W

    pltpu.sync_copy(idx_hbm.at[pl.ds(base, PER_W)], idx_v)
    pltpu.sync_copy(s_hbm, s_v)

    den = jnp.zeros((LANES,), jnp.float32)
    for t in range(PER_W // LANES):
        iv = idx_v[pl.ds(t * LANES, LANES)]
        sv = plsc.load_gather(s_v, [iv])
        ssel_v[pl.ds(t * LANES, LANES)] = sv
        den = den + sv
    den_v[...] = den

    def zero_body(j, _):
        acc_v[pl.ds(j * LANES, LANES)] = jnp.zeros((LANES,), jnp.float32)
        return 0
    lax.fori_loop(0, D_VECS, zero_body, 0)

    def chunk_body(k, _):
        pltpu.async_copy(tokens_hbm.at[idx_v.at[pl.ds(k * G, G)]],
                         rows_v, sem).wait()
        w = [plsc.load_gather(ssel_v, [jnp.full((LANES,), k * G + r, jnp.int32)])
             for r in range(G)]

        def col_body(j, _):
            sl = pl.ds(j * LANES, LANES)
            a = acc_v[sl]
            for r in range(G):
                a = a + w[r] * rows_v[r, sl]
            acc_v[sl] = a
            return 0
        lax.fori_loop(0, D_VECS, col_body, 0)
        return 0
    lax.fori_loop(0, N_CHUNK, chunk_body, 0)

    pltpu.sync_copy(acc_v, acc_out.at[wid])
    pltpu.sync_copy(den_v, den_out.at[wid])


@functools.partial(
    pl.kernel,
    mesh=_mesh,
    compiler_params=_params,
    out_type=jax.ShapeDtypeStruct((1, D), jnp.float32),
    scratch_types=[
        pltpu.VMEM((NW, COLS_W), jnp.float32),
        pltpu.VMEM((NW, LANES), jnp.float32),
        pltpu.VMEM((COLS_W,), jnp.float32),
    ],
)
def _combine(acc_hbm, den_hbm, out_hbm, cols_v, den_v, o_v):
    wid = lax.axis_index("s") * NC + lax.axis_index("c")
    c0 = wid * COLS_W

    pltpu.sync_copy(den_hbm, den_v)
    for r in range(NW):
        pltpu.sync_copy(acc_hbm.at[r, pl.ds(c0, COLS_W)], cols_v.at[r])

    den = jnp.zeros((LANES,), jnp.float32)
    for r in range(NW):
        den = den + den_v[r, :]
    dvec = jnp.full((LANES,), jnp.sum(den, axis=0) + 1e-6, jnp.float32)
    inv = jnp.ones((LANES,), jnp.float32) / dvec

    for j in range(COLS_W // LANES):
        sl = pl.ds(j * LANES, LANES)
        a = jnp.zeros((LANES,), jnp.float32)
        for r in range(NW):
            a = a + cols_v[r, sl]
        o_v[sl] = a * inv
    pltpu.sync_copy(o_v, out_hbm.at[0, pl.ds(c0, COLS_W)])


def kernel(tokens, s, idx):
    idx32 = idx.astype(jnp.int32)
    acc, den = _partial_sums(tokens, s, idx32)
    return _combine(acc, den)


# speedup vs baseline: 1.4893x; 1.4893x over previous
import functools

import jax
import jax.numpy as jnp
from jax import lax
from jax.experimental import pallas as pl
from jax.experimental.pallas import tpu as pltpu
from jax.experimental.pallas import tpu_sc as plsc

N_TOK = 8192
D = 4096
N_IDX = 4096
NC = 2
NS = 16
NW = NC * NS
PER_W = N_IDX // NW
G = 8
N_CHUNK = PER_W // G
LANES = 16
D_VECS = D // LANES
COLS_W = D // NW

_mesh = plsc.VectorSubcoreMesh(core_axis_name="c", subcore_axis_name="s")
_params = pltpu.CompilerParams(needs_layout_passes=False)


@functools.partial(
    pl.kernel,
    mesh=_mesh,
    compiler_params=_params,
    out_type=[
        jax.ShapeDtypeStruct((NW, D), jnp.float32),
        jax.ShapeDtypeStruct((NW, LANES), jnp.float32),
    ],
    scratch_types=[
        pltpu.VMEM((PER_W,), jnp.int32),
        pltpu.VMEM((N_TOK,), jnp.float32),
        pltpu.VMEM((PER_W,), jnp.float32),
        pltpu.VMEM((G, D), jnp.float32),
        pltpu.VMEM((G, D), jnp.float32),
        pltpu.VMEM((D,), jnp.float32),
        pltpu.VMEM((LANES,), jnp.float32),
        pltpu.SemaphoreType.DMA,
        pltpu.SemaphoreType.DMA,
    ],
)
def _partial_sums(tokens_hbm, s_hbm, idx_hbm, acc_out, den_out,
                  idx_v, s_v, ssel_v, rows0_v, rows1_v, acc_v, den_v,
                  sem0, sem1):
    wid = lax.axis_index("s") * NC + lax.axis_index("c")
    base = wid * PER_W

    pltpu.sync_copy(idx_hbm.at[pl.ds(base, PER_W)], idx_v)

    def gather_start(k, buf, sem):
        pltpu.async_copy(tokens_hbm.at[idx_v.at[pl.ds(k * G, G)]], buf, sem)

    def gather_wait(k, buf, sem):
        pltpu.make_async_copy(tokens_hbm.at[idx_v.at[pl.ds(k * G, G)]],
                              buf, sem).wait()

    gather_start(0, rows0_v, sem0)
    pltpu.sync_copy(s_hbm, s_v)

    den = jnp.zeros((LANES,), jnp.float32)
    for t in range(PER_W // LANES):
        iv = idx_v[pl.ds(t * LANES, LANES)]
        sv = plsc.load_gather(s_v, [iv])
        ssel_v[pl.ds(t * LANES, LANES)] = sv
        den = den + sv
    den_v[...] = den

    def zero_body(j, _):
        acc_v[pl.ds(j * LANES, LANES)] = jnp.zeros((LANES,), jnp.float32)
        return 0
    lax.fori_loop(0, D_VECS, zero_body, 0)

    def process(buf, k):
        w = [plsc.load_gather(ssel_v, [jnp.full((LANES,), k * G + r, jnp.int32)])
             for r in range(G)]

        def col_body(j, _):
            sl = pl.ds(j * LANES, LANES)
            a = acc_v[sl]
            for r in range(G):
                a = a + w[r] * buf[r, sl]
            acc_v[sl] = a
            return 0
        lax.fori_loop(0, D_VECS, col_body, 0)

    def body(t, _):
        k0 = 2 * t
        gather_start(k0 + 1, rows1_v, sem1)
        gather_wait(k0, rows0_v, sem0)
        process(rows0_v, k0)

        @pl.when(t < N_CHUNK // 2 - 1)
        def _():
            gather_start(k0 + 2, rows0_v, sem0)

        gather_wait(k0 + 1, rows1_v, sem1)
        process(rows1_v, k0 + 1)
        return 0
    lax.fori_loop(0, N_CHUNK // 2, body, 0)

    pltpu.sync_copy(acc_v, acc_out.at[wid])
    pltpu.sync_copy(den_v, den_out.at[wid])


@functools.partial(
    pl.kernel,
    mesh=_mesh,
    compiler_params=_params,
    out_type=jax.ShapeDtypeStruct((1, D), jnp.float32),
    scratch_types=[
        pltpu.VMEM((NW, COLS_W), jnp.float32),
        pltpu.VMEM((NW, LANES), jnp.float32),
        pltpu.VMEM((COLS_W,), jnp.float32),
        pltpu.SemaphoreType.DMA,
    ],
)
def _combine(acc_hbm, den_hbm, out_hbm, cols_v, den_v, o_v, sem):
    wid = lax.axis_index("s") * NC + lax.axis_index("c")
    c0 = wid * COLS_W

    for r in range(NW):
        pltpu.async_copy(acc_hbm.at[r, pl.ds(c0, COLS_W)], cols_v.at[r], sem)
    pltpu.sync_copy(den_hbm, den_v)
    pltpu.make_async_copy(acc_hbm.at[pl.ds(0, NW), pl.ds(0, COLS_W)], cols_v,
                          sem).wait()

    den = jnp.zeros((LANES,), jnp.float32)
    for r in range(NW):
        den = den + den_v[r, :]
    dvec = jnp.full((LANES,), jnp.sum(den, axis=0) + 1e-6, jnp.float32)
    inv = jnp.ones((LANES,), jnp.float32) / dvec

    for j in range(COLS_W // LANES):
        sl = pl.ds(j * LANES, LANES)
        a = jnp.zeros((LANES,), jnp.float32)
        for r in range(NW):
            a = a + cols_v[r, sl]
        o_v[sl] = a * inv
    pltpu.sync_copy(o_v, out_hbm.at[0, pl.ds(c0, COLS_W)])


def kernel(tokens, s, idx):
    idx32 = idx.astype(jnp.int32)
    acc, den = _partial_sums(tokens, s, idx32)
    return _combine(acc, den)


# speedup vs baseline: 1.6445x; 1.1042x over previous
import functools

import jax
import jax.numpy as jnp
from jax import lax
from jax.experimental import pallas as pl
from jax.experimental.pallas import tpu as pltpu
from jax.experimental.pallas import tpu_sc as plsc

N_TOK = 8192
D = 4096
N_IDX = 4096
NC = 2
NS = 16
NW = NC * NS
PER_W = N_IDX // NW
G = 8
N_CHUNK = PER_W // G
LANES = 16
D_VECS = D // LANES
COLS_W = D // NW

_mesh = plsc.VectorSubcoreMesh(core_axis_name="c", subcore_axis_name="s")
_params = pltpu.CompilerParams(needs_layout_passes=False)


@functools.partial(
    pl.kernel,
    mesh=_mesh,
    compiler_params=_params,
    out_type=[
        jax.ShapeDtypeStruct((NW, D), jnp.float32),
        jax.ShapeDtypeStruct((NW, LANES), jnp.float32),
    ],
    scratch_types=[
        pltpu.VMEM((PER_W,), jnp.int32),
        pltpu.VMEM((PER_W,), jnp.float32),
        pltpu.VMEM((G, D), jnp.float32),
        pltpu.VMEM((G, D), jnp.float32),
        pltpu.VMEM((D,), jnp.float32),
        pltpu.VMEM((LANES,), jnp.float32),
        pltpu.SemaphoreType.DMA,
        pltpu.SemaphoreType.DMA,
        pltpu.SemaphoreType.DMA,
    ],
)
def _partial_sums(tokens_hbm, s_hbm, idx_hbm, acc_out, den_out,
                  idx_v, ssel_v, rows0_v, rows1_v, acc_v, den_v,
                  sem0, sem1, sems):
    wid = lax.axis_index("s") * NC + lax.axis_index("c")
    base = wid * PER_W

    pltpu.sync_copy(idx_hbm.at[pl.ds(base, PER_W)], idx_v)

    def gather_start(k, buf, sem):
        pltpu.async_copy(tokens_hbm.at[idx_v.at[pl.ds(k * G, G)]], buf, sem)

    def gather_wait(k, buf, sem):
        pltpu.make_async_copy(tokens_hbm.at[idx_v.at[pl.ds(k * G, G)]],
                              buf, sem).wait()

    gather_start(0, rows0_v, sem0)
    pltpu.async_copy(s_hbm.at[idx_v], ssel_v, sems).wait()

    den = jnp.zeros((LANES,), jnp.float32)
    for t in range(PER_W // LANES):
        den = den + ssel_v[pl.ds(t * LANES, LANES)]
    den_v[...] = den

    def zero_body(j, _):
        acc_v[pl.ds(j * LANES, LANES)] = jnp.zeros((LANES,), jnp.float32)
        return 0
    lax.fori_loop(0, D_VECS, zero_body, 0, unroll=4)

    def process(buf, k):
        w = [plsc.load_gather(ssel_v, [jnp.full((LANES,), k * G + r, jnp.int32)])
             for r in range(G)]

        def col_body(j, _):
            sl = pl.ds(j * LANES, LANES)
            a = acc_v[sl]
            for r in range(G):
                a = a + w[r] * buf[r, sl]
            acc_v[sl] = a
            return 0
        lax.fori_loop(0, D_VECS, col_body, 0, unroll=4)

    def body(t, _):
        k0 = 2 * t
        gather_start(k0 + 1, rows1_v, sem1)
        gather_wait(k0, rows0_v, sem0)
        process(rows0_v, k0)

        @pl.when(t < N_CHUNK // 2 - 1)
        def _():
            gather_start(k0 + 2, rows0_v, sem0)

        gather_wait(k0 + 1, rows1_v, sem1)
        process(rows1_v, k0 + 1)
        return 0
    lax.fori_loop(0, N_CHUNK // 2, body, 0)

    pltpu.sync_copy(acc_v, acc_out.at[wid])
    pltpu.sync_copy(den_v, den_out.at[wid])


def _combine_tc(acc_ref, den_ref, out_ref):
    den = jnp.sum(den_ref[...]) + 1e-6
    out_ref[...] = jnp.sum(acc_ref[...], axis=0, keepdims=True) / den


def kernel(tokens, s, idx):
    idx32 = idx.astype(jnp.int32)
    acc, den = _partial_sums(tokens, s, idx32)
    return pl.pallas_call(
        _combine_tc,
        out_shape=jax.ShapeDtypeStruct((1, D), jnp.float32),
    )(acc, den)
